# Initial kernel scaffold; baseline (speedup 1.0000x reference)
import jax
import jax.numpy as jnp
from jax.experimental import pallas as pl


def kernel(x, batch, weight, running_prior):
    raise NotImplementedError("write your pallas kernel here")


# speedup vs baseline: 1.0013x; 1.0013x over previous
import functools

import jax
import jax.numpy as jnp
from jax import lax
from jax.experimental import pallas as pl
from jax.experimental.pallas import tpu as pltpu
from jax.experimental.pallas import tpu_sc as plsc

N_EMB = 512
E_DIM = 64
ALPHA = 0.001
EPS = 1e-8
KB = 8


def _prep_body(xf_ref, w_ref, rp_ref,
               w0t_ref, q1t_ref, q2s_ref, pi_ref, idx_ref, sp_ref):
    n = xf_ref.shape[0]
    seg = 256
    f32 = jnp.float32
    xf = xf_ref[...]
    w = w_ref[...]
    ones_row = jnp.ones((1, E_DIM), f32)

    x2c = jnp.sum(xf * xf, axis=1, keepdims=True)
    w2c = jnp.sum(w * w, axis=1, keepdims=True)
    x2r = lax.dot_general(ones_row, xf * xf, (((1,), (1,)), ((), ())),
                          preferred_element_type=f32)
    w2r = lax.dot_general(ones_row, w * w, (((1,), (1,)), ((), ())),
                          preferred_element_type=f32)
    mm = lax.dot_general(xf, w, (((1,), (1,)), ((), ())),
                         preferred_element_type=f32)
    mmt = lax.dot_general(w, xf, (((1,), (1,)), ((), ())),
                          preferred_element_type=f32)
    d = (x2c + w2r) - 2.0 * mm
    dt = (w2c + x2r) - 2.0 * mmt

    mn = jnp.min(d, axis=1, keepdims=True)
    iota_k = lax.broadcasted_iota(jnp.int32, (n, N_EMB), 1)
    cand = jnp.where(d == mn, iota_k, jnp.int32(N_EMB))
    idxc = jnp.min(cand, axis=1, keepdims=True)
    idx_ref[...] = idxc
    onehot = (iota_k == idxc).astype(f32)
    counts = jnp.sum(onehot, axis=0, keepdims=True)
    p = counts * (1.0 / n)
    rp = rp_ref[...]
    kl = jnp.sum(p * (jnp.log(p + 1e-10) - jnp.log(rp + 1e-10)))
    mse = jnp.sum(mn) * (1.0 / (n * E_DIM))
    sp_ref[...] = jnp.full((1, 1), 0.0, f32) + (1.25 * mse + 1.0 * kl)

    mx = jnp.max(d, axis=1, keepdims=True)
    sh = d - mx
    lse = jnp.log(jnp.sum(jnp.exp(sh), axis=1, keepdims=True))
    q = sh - lse
    mxr = jnp.max(dt, axis=0, keepdims=True)
    sht = dt - mxr
    lser = jnp.log(jnp.sum(jnp.exp(sht), axis=0, keepdims=True))
    qt = sht - lser

    lane_f = lax.broadcasted_iota(jnp.int32, (1, n), 1) % seg
    w0t_ref[...] = jnp.where(lane_f < seg - 2, qt, 0.0)
    q1t = jnp.concatenate([qt[:, 1:], qt[:, :1]], axis=1)
    q1t_ref[...] = q1t
    q2s_ref[...] = jnp.concatenate([q[2:], q[:2]], axis=0)

    qpt = jnp.where(lane_f < seg - 1, qt, 0.0)
    cp = lax.dot_general(qpt, q1t, (((1,), (1,)), ((), ())),
                         preferred_element_type=f32)
    pi_ref[...] = cp * (1.0 / (jnp.sum(cp) + EPS))


def _h2_body(w0t_ref, pi_ref, q1t_ref, q2s_ref, sp_ref, out_ref, acc_ref):
    i = pl.program_id(0)
    nb = pl.num_programs(0)
    f32 = jnp.float32

    @pl.when(i == 0)
    def _init():
        acc_ref[...] = jnp.zeros((1, 1), f32)

    q1t = q1t_ref[...]
    q2s = q2s_ref[...]
    for kk in range(KB):
        wrow = w0t_ref[kk:kk + 1, :]
        bmat = q1t * wrow
        c = lax.dot_general(bmat, q2s, (((1,), (0,)), ((), ())),
                            preferred_element_type=f32)
        cs = c + ALPHA
        s0 = jnp.sum(cs, axis=1, keepdims=True)
        t = cs * (1.0 / (s0 + EPS))
        ent = t * jnp.log(t + EPS)
        hrow = -jnp.sum(ent, axis=1, keepdims=True)
        pirow = pi_ref[kk:kk + 1, :]
        acc_ref[...] += lax.dot_general(
            pirow, hrow, (((1,), (0,)), ((), ())),
            preferred_element_type=f32)

    @pl.when(i == nb - 1)
    def _fin():
        out_ref[...] = sp_ref[...] + 0.1 * acc_ref[...]


def _sc_gather(weight, idx):
    dpad = 128
    info = plsc.get_sparse_core_info()
    nw = info.num_cores * info.num_subcores
    bpw = N_EMB // nw
    mesh = plsc.VectorSubcoreMesh(core_axis_name="c", subcore_axis_name="s")

    @functools.partial(
        pl.kernel, mesh=mesh,
        out_type=jax.ShapeDtypeStruct((N_EMB, dpad), jnp.float32),
        scratch_types=[
            pltpu.VMEM((bpw,), jnp.int32),
            pltpu.VMEM((bpw, dpad), jnp.float32),
            pltpu.SemaphoreType.DMA,
        ],
    )
    def gk(table_hbm, idx_hbm, out_hbm, idx_v, rows_v, sem):
        wid = lax.axis_index("s") * info.num_cores + lax.axis_index("c")
        base = wid * bpw
        pltpu.sync_copy(idx_hbm.at[pl.ds(base, bpw)], idx_v)
        pltpu.async_copy(table_hbm.at[idx_v], rows_v, sem).wait()
        pltpu.sync_copy(rows_v, out_hbm.at[pl.ds(base, bpw)])

    wp = jnp.pad(weight, ((0, 0), (0, dpad - E_DIM)))
    return gk(wp, idx)[:, :E_DIM]


def kernel(x, batch, weight, running_prior):
    n = x.shape[0] * x.shape[1]
    xf = x.reshape(n, E_DIM)
    rp = running_prior.reshape(1, N_EMB)

    f32 = jnp.float32
    w0t, q1t, q2s, pi, idx2d, sp = pl.pallas_call(
        _prep_body,
        out_shape=[
            jax.ShapeDtypeStruct((N_EMB, n), f32),
            jax.ShapeDtypeStruct((N_EMB, n), f32),
            jax.ShapeDtypeStruct((n, N_EMB), f32),
            jax.ShapeDtypeStruct((N_EMB, N_EMB), f32),
            jax.ShapeDtypeStruct((n, 1), jnp.int32),
            jax.ShapeDtypeStruct((1, 1), f32),
        ],
    )(xf, weight, rp)

    quant = _sc_gather(weight, idx2d.reshape(n))

    nb = N_EMB // KB
    total = pl.pallas_call(
        _h2_body,
        grid=(nb,),
        in_specs=[
            pl.BlockSpec((KB, n), lambda i: (i, 0)),
            pl.BlockSpec((KB, N_EMB), lambda i: (i, 0)),
            pl.BlockSpec((N_EMB, n), lambda i: (0, 0)),
            pl.BlockSpec((n, N_EMB), lambda i: (0, 0)),
            pl.BlockSpec((1, 1), lambda i: (0, 0)),
        ],
        out_specs=pl.BlockSpec((1, 1), lambda i: (0, 0)),
        out_shape=jax.ShapeDtypeStruct((1, 1), f32),
        scratch_shapes=[pltpu.VMEM((1, 1), f32)],
    )(w0t, pi, q1t, q2s, sp)

    return quant.reshape(x.shape), total[0, 0]


# speedup vs baseline: 1.0045x; 1.0032x over previous
import functools

import jax
import jax.numpy as jnp
from jax import lax
from jax.experimental import pallas as pl
from jax.experimental.pallas import tpu as pltpu
from jax.experimental.pallas import tpu_sc as plsc

N_EMB = 512
E_DIM = 64
ALPHA = 0.001
EPS = 1e-8
KB = 8


def _prep_body(xf_ref, w_ref, rp_ref,
               w0t_ref, q1t_ref, q2s_ref, pi_ref, idx_ref, sp_ref):
    n = xf_ref.shape[0]
    seg = 256
    f32 = jnp.float32
    xf = xf_ref[...]
    w = w_ref[...]
    ones_row = jnp.ones((1, E_DIM), f32)

    x2c = jnp.sum(xf * xf, axis=1, keepdims=True)
    w2c = jnp.sum(w * w, axis=1, keepdims=True)
    x2r = lax.dot_general(ones_row, xf * xf, (((1,), (1,)), ((), ())),
                          preferred_element_type=f32)
    w2r = lax.dot_general(ones_row, w * w, (((1,), (1,)), ((), ())),
                          preferred_element_type=f32)
    mm = lax.dot_general(xf, w, (((1,), (1,)), ((), ())),
                         preferred_element_type=f32)
    mmt = lax.dot_general(w, xf, (((1,), (1,)), ((), ())),
                          preferred_element_type=f32)
    d = (x2c + w2r) - 2.0 * mm
    dt = (w2c + x2r) - 2.0 * mmt

    mn = jnp.min(d, axis=1, keepdims=True)
    iota_k = lax.broadcasted_iota(jnp.int32, (n, N_EMB), 1)
    cand = jnp.where(d == mn, iota_k, jnp.int32(N_EMB))
    idxc = jnp.min(cand, axis=1, keepdims=True)
    idx_ref[...] = idxc
    onehot = (iota_k == idxc).astype(f32)
    counts = jnp.sum(onehot, axis=0, keepdims=True)
    p = counts * (1.0 / n)
    rp = rp_ref[...]
    kl = jnp.sum(p * (jnp.log(p + 1e-10) - jnp.log(rp + 1e-10)))
    mse = jnp.sum(mn) * (1.0 / (n * E_DIM))
    sp_ref[...] = jnp.full((1, 1), 0.0, f32) + (1.25 * mse + 1.0 * kl)

    mx = jnp.max(d, axis=1, keepdims=True)
    sh = d - mx
    lse = jnp.log(jnp.sum(jnp.exp(sh), axis=1, keepdims=True))
    q = sh - lse
    mxr = jnp.max(dt, axis=0, keepdims=True)
    sht = dt - mxr
    lser = jnp.log(jnp.sum(jnp.exp(sht), axis=0, keepdims=True))
    qt = sht - lser

    lane_f = lax.broadcasted_iota(jnp.int32, (1, n), 1) % seg
    w0t_ref[...] = jnp.where(lane_f < seg - 2, qt, 0.0).astype(jnp.bfloat16)
    q1t = jnp.concatenate([qt[:, 1:], qt[:, :1]], axis=1)
    q1t_ref[...] = q1t.astype(jnp.bfloat16)
    q2s_ref[...] = jnp.concatenate([q[2:], q[:2]], axis=0).astype(jnp.bfloat16)

    qpt = jnp.where(lane_f < seg - 1, qt, 0.0)
    cp = lax.dot_general(qpt, q1t, (((1,), (1,)), ((), ())),
                         preferred_element_type=f32)
    pi_ref[...] = cp * (1.0 / (jnp.sum(cp) + EPS))


def _h2_body(w0t_ref, pi_ref, q1t_ref, q2s_ref, sp_ref, out_ref, acc_ref):
    i = pl.program_id(0)
    nb = pl.num_programs(0)
    f32 = jnp.float32

    @pl.when(i == 0)
    def _init():
        acc_ref[...] = jnp.zeros((1, 1), f32)

    q1t = q1t_ref[...]
    q2s = q2s_ref[...]
    for kk in range(KB):
        wrow = w0t_ref[kk:kk + 1, :]
        bmat = q1t * wrow
        c = lax.dot_general(bmat, q2s, (((1,), (0,)), ((), ())),
                            preferred_element_type=f32)
        cs = c + ALPHA
        s0 = jnp.sum(cs, axis=1, keepdims=True)
        t = cs * (1.0 / (s0 + EPS))
        ent = t * jnp.log(t + EPS)
        hrow = -jnp.sum(ent, axis=1, keepdims=True)
        pirow = pi_ref[kk:kk + 1, :]
        acc_ref[...] += lax.dot_general(
            pirow, hrow, (((1,), (0,)), ((), ())),
            preferred_element_type=f32)

    @pl.when(i == nb - 1)
    def _fin():
        out_ref[...] = sp_ref[...] + 0.1 * acc_ref[...]


def _sc_gather(weight, idx):
    dpad = 128
    info = plsc.get_sparse_core_info()
    nw = info.num_cores * info.num_subcores
    bpw = N_EMB // nw
    mesh = plsc.VectorSubcoreMesh(core_axis_name="c", subcore_axis_name="s")

    @functools.partial(
        pl.kernel, mesh=mesh,
        out_type=jax.ShapeDtypeStruct((N_EMB, dpad), jnp.float32),
        scratch_types=[
            pltpu.VMEM((bpw,), jnp.int32),
            pltpu.VMEM((bpw, dpad), jnp.float32),
            pltpu.SemaphoreType.DMA,
        ],
    )
    def gk(table_hbm, idx_hbm, out_hbm, idx_v, rows_v, sem):
        wid = lax.axis_index("s") * info.num_cores + lax.axis_index("c")
        base = wid * bpw
        pltpu.sync_copy(idx_hbm.at[pl.ds(base, bpw)], idx_v)
        pltpu.async_copy(table_hbm.at[idx_v], rows_v, sem).wait()
        pltpu.sync_copy(rows_v, out_hbm.at[pl.ds(base, bpw)])

    wp = jnp.pad(weight, ((0, 0), (0, dpad - E_DIM)))
    return gk(wp, idx)[:, :E_DIM]


def kernel(x, batch, weight, running_prior):
    n = x.shape[0] * x.shape[1]
    xf = x.reshape(n, E_DIM)
    rp = running_prior.reshape(1, N_EMB)

    f32 = jnp.float32
    w0t, q1t, q2s, pi, idx2d, sp = pl.pallas_call(
        _prep_body,
        out_shape=[
            jax.ShapeDtypeStruct((N_EMB, n), jnp.bfloat16),
            jax.ShapeDtypeStruct((N_EMB, n), jnp.bfloat16),
            jax.ShapeDtypeStruct((n, N_EMB), jnp.bfloat16),
            jax.ShapeDtypeStruct((N_EMB, N_EMB), f32),
            jax.ShapeDtypeStruct((n, 1), jnp.int32),
            jax.ShapeDtypeStruct((1, 1), f32),
        ],
    )(xf, weight, rp)

    quant = _sc_gather(weight, idx2d.reshape(n))

    nb = N_EMB // KB
    total = pl.pallas_call(
        _h2_body,
        grid=(nb,),
        in_specs=[
            pl.BlockSpec((KB, n), lambda i: (i, 0)),
            pl.BlockSpec((KB, N_EMB), lambda i: (i, 0)),
            pl.BlockSpec((N_EMB, n), lambda i: (0, 0)),
            pl.BlockSpec((n, N_EMB), lambda i: (0, 0)),
            pl.BlockSpec((1, 1), lambda i: (0, 0)),
        ],
        out_specs=pl.BlockSpec((1, 1), lambda i: (0, 0)),
        out_shape=jax.ShapeDtypeStruct((1, 1), f32),
        scratch_shapes=[pltpu.VMEM((1, 1), f32)],
    )(w0t, pi, q1t, q2s, sp)

    return quant.reshape(x.shape), total[0, 0]
